# Initial kernel scaffold; baseline (speedup 1.0000x reference)
import jax
import jax.numpy as jnp
from jax.experimental import pallas as pl


def kernel(x, edge_index, edge_attr, prelu_a, W_enc, emb1, emb2, W1, b1, W2, b2):
    raise NotImplementedError("write your pallas kernel here")


# speedup vs baseline: 8.2347x; 8.2347x over previous
import jax
import jax.numpy as jnp
from jax import lax
from jax.experimental import pallas as pl
from jax.experimental.pallas import tpu as pltpu
from jax.experimental.pallas import tpu_sc as plsc

N_NODES = 10000
HIDDEN = 128
HH = HIDDEN // 2
N_EDGES = 320000
NC = 2
NS = 16
NW = NC * NS
EPW = N_EDGES // NW
B = 128
NB = -(-EPW // B)
EPAD = NB * B
NPADE = EPAD - EPW
ACC_ROWS = 10240
ZROWS = ACC_ROWS // NS
OROWS = 624


def _enc_body(a_ref, x_ref, w_ref, o0_ref, o1_ref):
    xa = x_ref[...]
    h = jnp.where(xa > 0, xa, a_ref[...] * xa)
    res = lax.dot_general(h, w_ref[...], (((1,), (1,)), ((), ())),
                          preferred_element_type=jnp.float32)
    o0_ref[...] = res[:, :HH]
    o1_ref[...] = res[:, HH:]


def _encode(x, prelu_a, W_enc):
    grid = 10
    rb = N_NODES // grid
    return pl.pallas_call(
        _enc_body,
        grid=(grid,),
        in_specs=[
            pl.BlockSpec((1, 1), lambda i: (0, 0)),
            pl.BlockSpec((rb, HIDDEN), lambda i: (i, 0)),
            pl.BlockSpec((HIDDEN, HIDDEN), lambda i: (0, 0)),
        ],
        out_specs=[
            pl.BlockSpec((rb, HH), lambda i: (i, 0)),
            pl.BlockSpec((rb, HH), lambda i: (i, 0)),
        ],
        out_shape=[
            jax.ShapeDtypeStruct((N_NODES, HH), jnp.float32),
            jax.ShapeDtypeStruct((N_NODES, HH), jnp.float32),
        ],
    )(prelu_a.reshape(1, 1), x, W_enc)


def _sc_body(h0_hbm, h1_hbm, src_hbm, dst_hbm, cmb_hbm, emb_hbm, out_hbm,
             idx_src, idx_dst, idx_cmb, arows, brows, zbuf,
             emb_sh, accum_sh, sem_a, sem_b):
    c = lax.axis_index("c")
    s = lax.axis_index("s")
    wid = s * NC + c

    zero16 = jnp.zeros((16,), jnp.float32)
    for i in range(16):
        for j in range(HH // 16):
            zbuf[i, pl.ds(j * 16, 16)] = zero16

    pltpu.sync_copy(src_hbm.at[wid], idx_src)
    pltpu.sync_copy(dst_hbm.at[wid], idx_dst)
    pltpu.sync_copy(cmb_hbm.at[wid], idx_cmb)

    for p, h_hbm in ((0, h0_hbm), (1, h1_hbm)):
        for t in range(ZROWS // 16):
            pltpu.sync_copy(zbuf, accum_sh.at[pl.ds(s * ZROWS + t * 16, 16)])

        @pl.when(s == 0)
        def _():
            pltpu.sync_copy(emb_hbm.at[p], emb_sh)

        plsc.subcore_barrier()

        def batch(b, carry):
            ga = pltpu.async_copy(h_hbm.at[idx_src.at[b]], arows, sem_a)
            gb = pltpu.async_copy(emb_sh.at[idx_cmb.at[b]], brows, sem_b)
            ga.wait()
            gb.wait()
            pltpu.sync_copy(arows, accum_sh.at[idx_dst.at[b]], add=True)
            pltpu.sync_copy(brows, accum_sh.at[idx_dst.at[b]], add=True)
            return carry

        lax.fori_loop(0, NB, batch, 0)

        plsc.subcore_barrier()

        pltpu.sync_copy(accum_sh.at[pl.ds(s * OROWS, OROWS)],
                        out_hbm.at[c].at[p].at[pl.ds(s * OROWS, OROWS)])

        @pl.when(s == NS - 1)
        def _():
            tail = N_NODES - NS * OROWS
            pltpu.sync_copy(accum_sh.at[pl.ds(NS * OROWS, tail)],
                            out_hbm.at[c].at[p].at[pl.ds(NS * OROWS, tail)])

        plsc.subcore_barrier()


def _aggregate(h0, h1, src3, dst3, cmb3, emb2x16):
    mesh = plsc.VectorSubcoreMesh(core_axis_name="c", subcore_axis_name="s")
    return pl.kernel(
        _sc_body,
        out_type=jax.ShapeDtypeStruct((NC, 2, N_NODES, HH), jnp.float32),
        mesh=mesh,
        compiler_params=pltpu.CompilerParams(use_tc_tiling_on_sc=False),
        scratch_types=[
            pltpu.VMEM((NB, B), jnp.int32),
            pltpu.VMEM((NB, B), jnp.int32),
            pltpu.VMEM((NB, B), jnp.int32),
            pltpu.VMEM((B, HH), jnp.float32),
            pltpu.VMEM((B, HH), jnp.float32),
            pltpu.VMEM((16, HH), jnp.float32),
            pltpu.VMEM_SHARED((16, HH), jnp.float32),
            pltpu.VMEM_SHARED((ACC_ROWS, HH), jnp.float32),
            pltpu.SemaphoreType.DMA,
            pltpu.SemaphoreType.DMA,
        ],
    )(h0, h1, src3, dst3, cmb3, emb2x16)


def _mlp_body(p00_ref, p01_ref, p10_ref, p11_ref, h0_ref, h1_ref, sv_ref,
              w1_ref, b1_ref, w2_ref, b2_ref, o_ref):
    sv = sv_ref[...]
    lo = p00_ref[...] + p10_ref[...] + h0_ref[...] + sv[:, :HH]
    hi = p01_ref[...] + p11_ref[...] + h1_ref[...] + sv[:, HH:]
    aggr = jnp.concatenate([lo, hi], axis=1)
    z = lax.dot_general(aggr, w1_ref[...], (((1,), (1,)), ((), ())),
                        preferred_element_type=jnp.float32) + b1_ref[...]
    z = jnp.maximum(z, 0.0)
    o_ref[...] = lax.dot_general(z, w2_ref[...], (((1,), (1,)), ((), ())),
                                 preferred_element_type=jnp.float32) + b2_ref[...]


def _mlp(parts, h0, h1, selfvec, W1, b1, W2, b2):
    grid = 10
    rb = N_NODES // grid
    H2 = 2 * HIDDEN
    return pl.pallas_call(
        _mlp_body,
        grid=(grid,),
        in_specs=[
            pl.BlockSpec((rb, HH), lambda i: (i, 0)),
            pl.BlockSpec((rb, HH), lambda i: (i, 0)),
            pl.BlockSpec((rb, HH), lambda i: (i, 0)),
            pl.BlockSpec((rb, HH), lambda i: (i, 0)),
            pl.BlockSpec((rb, HH), lambda i: (i, 0)),
            pl.BlockSpec((rb, HH), lambda i: (i, 0)),
            pl.BlockSpec((1, HIDDEN), lambda i: (0, 0)),
            pl.BlockSpec((H2, HIDDEN), lambda i: (0, 0)),
            pl.BlockSpec((1, H2), lambda i: (0, 0)),
            pl.BlockSpec((HIDDEN, H2), lambda i: (0, 0)),
            pl.BlockSpec((1, HIDDEN), lambda i: (0, 0)),
        ],
        out_specs=pl.BlockSpec((rb, HIDDEN), lambda i: (i, 0)),
        out_shape=jax.ShapeDtypeStruct((N_NODES, HIDDEN), jnp.float32),
    )(parts[0, 0], parts[0, 1], parts[1, 0], parts[1, 1], h0, h1, selfvec,
      W1, b1.reshape(1, H2), W2, b2.reshape(1, HIDDEN))


def kernel(x, edge_index, edge_attr, prelu_a, W_enc, emb1, emb2, W1, b1, W2,
           b2):
    src = edge_index[0].astype(jnp.int32)
    dst = edge_index[1].astype(jnp.int32)
    cmb = (edge_attr[:, 0] * 3 + edge_attr[:, 1]).astype(jnp.int32)

    pi = jnp.arange(NPADE, dtype=jnp.int32)[None, :]
    wi = jnp.arange(NW, dtype=jnp.int32)[:, None]
    pad_src = (pi * 89 + wi * 313) % N_NODES
    pad_dst = N_NODES + (pi + wi * 7) % (ACC_ROWS - N_NODES)
    pad_cmb = (pi + wi) % 16

    src3 = jnp.concatenate([src.reshape(NW, EPW), pad_src], axis=1)
    dst3 = jnp.concatenate([dst.reshape(NW, EPW), pad_dst], axis=1)
    cmb3 = jnp.concatenate([cmb.reshape(NW, EPW), pad_cmb], axis=1)
    src3 = src3.reshape(NW, NB, B)
    dst3 = dst3.reshape(NW, NB, B)
    cmb3 = cmb3.reshape(NW, NB, B)

    emb9 = (emb1[:3, None, :] + emb2[None, :3, :]).reshape(9, HIDDEN)
    emb16 = jnp.concatenate([emb9, jnp.zeros((7, HIDDEN), jnp.float32)],
                            axis=0)
    emb2x16 = jnp.stack([emb16[:, :HH], emb16[:, HH:]], axis=0)
    selfvec = (emb1[4] + emb2[0]).reshape(1, HIDDEN)

    h0, h1 = _encode(x, prelu_a, W_enc)
    parts = _aggregate(h0, h1, src3, dst3, cmb3, emb2x16)
    return _mlp(parts, h0, h1, selfvec, W1, b1, W2, b2)


# speedup vs baseline: 9.8201x; 1.1925x over previous
import jax
import jax.numpy as jnp
from jax import lax
from jax.experimental import pallas as pl
from jax.experimental.pallas import tpu as pltpu
from jax.experimental.pallas import tpu_sc as plsc

N_NODES = 10000
HIDDEN = 128
HH = HIDDEN // 2
N_EDGES = 320000
NC = 2
NS = 16
NW = NC * NS
EPW = N_EDGES // NW
B = 128
NB = -(-EPW // B)
EPAD = NB * B
NPADE = EPAD - EPW
ACC_ROWS = 10240
ZROWS = ACC_ROWS // NS
OROWS = 624


def _enc_body(a_ref, x_ref, w_ref, o0_ref, o1_ref):
    xa = x_ref[...]
    h = jnp.where(xa > 0, xa, a_ref[...] * xa)
    res = lax.dot_general(h, w_ref[...], (((1,), (1,)), ((), ())),
                          preferred_element_type=jnp.float32)
    o0_ref[...] = res[:, :HH]
    o1_ref[...] = res[:, HH:]


def _encode(x, prelu_a, W_enc):
    grid = 10
    rb = N_NODES // grid
    return pl.pallas_call(
        _enc_body,
        grid=(grid,),
        in_specs=[
            pl.BlockSpec((1, 1), lambda i: (0, 0)),
            pl.BlockSpec((rb, HIDDEN), lambda i: (i, 0)),
            pl.BlockSpec((HIDDEN, HIDDEN), lambda i: (0, 0)),
        ],
        out_specs=[
            pl.BlockSpec((rb, HH), lambda i: (i, 0)),
            pl.BlockSpec((rb, HH), lambda i: (i, 0)),
        ],
        out_shape=[
            jax.ShapeDtypeStruct((N_NODES, HH), jnp.float32),
            jax.ShapeDtypeStruct((N_NODES, HH), jnp.float32),
        ],
    )(prelu_a.reshape(1, 1), x, W_enc)


def _sc_body(h0_hbm, h1_hbm, src_hbm, dst_hbm, cmb_hbm, emb_hbm, out_hbm,
             idx_src, idx_dst, idx_cmb, arows0, brows0, arows1, brows1, zbuf,
             emb_sh, accum_sh, sem_a0, sem_b0, sem_a1, sem_b1):
    c = lax.axis_index("c")
    s = lax.axis_index("s")
    wid = s * NC + c

    zero16 = jnp.zeros((16,), jnp.float32)
    for i in range(16):
        for j in range(HH // 16):
            zbuf[i, pl.ds(j * 16, 16)] = zero16

    pltpu.sync_copy(src_hbm.at[wid], idx_src)
    pltpu.sync_copy(dst_hbm.at[wid], idx_dst)
    pltpu.sync_copy(cmb_hbm.at[wid], idx_cmb)

    for p, h_hbm in ((0, h0_hbm), (1, h1_hbm)):
        for t in range(ZROWS // 16):
            pltpu.sync_copy(zbuf, accum_sh.at[pl.ds(s * ZROWS + t * 16, 16)])

        @pl.when(s == 0)
        def _():
            pltpu.sync_copy(emb_hbm.at[p], emb_sh)

        plsc.subcore_barrier()

        def issue(b, ar, br, sa, sb):
            ga = pltpu.async_copy(h_hbm.at[idx_src.at[b]], ar, sa)
            gb = pltpu.async_copy(emb_sh.at[idx_cmb.at[b]], br, sb)
            return ga, gb

        def drain_and_scatter(b, ga, gb, ar, br):
            ga.wait()
            gb.wait()
            pltpu.sync_copy(ar, accum_sh.at[idx_dst.at[b]], add=True)
            pltpu.sync_copy(br, accum_sh.at[idx_dst.at[b]], add=True)

        issue(0, arows0, brows0, sem_a0, sem_b0)

        def batch2(b, carry):
            g1 = issue(b + 1, arows1, brows1, sem_a1, sem_b1)
            g0 = (pltpu.make_async_copy(h_hbm.at[idx_src.at[b]], arows0,
                                        sem_a0),
                  pltpu.make_async_copy(emb_sh.at[idx_cmb.at[b]], brows0,
                                        sem_b0))
            drain_and_scatter(b, g0[0], g0[1], arows0, brows0)
            issue(b + 2, arows0, brows0, sem_a0, sem_b0)
            drain_and_scatter(b + 1, g1[0], g1[1], arows1, brows1)
            return carry

        lax.fori_loop(0, (NB - 1) // 2, lambda k, c: batch2(2 * k, c), 0,
                      unroll=False)
        gl = (pltpu.make_async_copy(h_hbm.at[idx_src.at[NB - 1]], arows0,
                                    sem_a0),
              pltpu.make_async_copy(emb_sh.at[idx_cmb.at[NB - 1]], brows0,
                                    sem_b0))
        drain_and_scatter(NB - 1, gl[0], gl[1], arows0, brows0)

        plsc.subcore_barrier()

        pltpu.sync_copy(accum_sh.at[pl.ds(s * OROWS, OROWS)],
                        out_hbm.at[c].at[p].at[pl.ds(s * OROWS, OROWS)])

        @pl.when(s == NS - 1)
        def _():
            tail = N_NODES - NS * OROWS
            pltpu.sync_copy(accum_sh.at[pl.ds(NS * OROWS, tail)],
                            out_hbm.at[c].at[p].at[pl.ds(NS * OROWS, tail)])

        plsc.subcore_barrier()


def _aggregate(h0, h1, src3, dst3, cmb3, emb2x16):
    mesh = plsc.VectorSubcoreMesh(core_axis_name="c", subcore_axis_name="s")
    return pl.kernel(
        _sc_body,
        out_type=jax.ShapeDtypeStruct((NC, 2, N_NODES, HH), jnp.float32),
        mesh=mesh,
        compiler_params=pltpu.CompilerParams(use_tc_tiling_on_sc=False),
        scratch_types=[
            pltpu.VMEM((NB, B), jnp.int32),
            pltpu.VMEM((NB, B), jnp.int32),
            pltpu.VMEM((NB, B), jnp.int32),
            pltpu.VMEM((B, HH), jnp.float32),
            pltpu.VMEM((B, HH), jnp.float32),
            pltpu.VMEM((B, HH), jnp.float32),
            pltpu.VMEM((B, HH), jnp.float32),
            pltpu.VMEM((16, HH), jnp.float32),
            pltpu.VMEM_SHARED((16, HH), jnp.float32),
            pltpu.VMEM_SHARED((ACC_ROWS, HH), jnp.float32),
            pltpu.SemaphoreType.DMA,
            pltpu.SemaphoreType.DMA,
            pltpu.SemaphoreType.DMA,
            pltpu.SemaphoreType.DMA,
        ],
    )(h0, h1, src3, dst3, cmb3, emb2x16)


def _mlp_body(p00_ref, p01_ref, p10_ref, p11_ref, h0_ref, h1_ref, sv_ref,
              w1_ref, b1_ref, w2_ref, b2_ref, o_ref):
    sv = sv_ref[...]
    lo = p00_ref[...] + p10_ref[...] + h0_ref[...] + sv[:, :HH]
    hi = p01_ref[...] + p11_ref[...] + h1_ref[...] + sv[:, HH:]
    aggr = jnp.concatenate([lo, hi], axis=1)
    z = lax.dot_general(aggr, w1_ref[...], (((1,), (1,)), ((), ())),
                        preferred_element_type=jnp.float32) + b1_ref[...]
    z = jnp.maximum(z, 0.0)
    o_ref[...] = lax.dot_general(z, w2_ref[...], (((1,), (1,)), ((), ())),
                                 preferred_element_type=jnp.float32) + b2_ref[...]


def _mlp(parts, h0, h1, selfvec, W1, b1, W2, b2):
    grid = 10
    rb = N_NODES // grid
    H2 = 2 * HIDDEN
    return pl.pallas_call(
        _mlp_body,
        grid=(grid,),
        in_specs=[
            pl.BlockSpec((rb, HH), lambda i: (i, 0)),
            pl.BlockSpec((rb, HH), lambda i: (i, 0)),
            pl.BlockSpec((rb, HH), lambda i: (i, 0)),
            pl.BlockSpec((rb, HH), lambda i: (i, 0)),
            pl.BlockSpec((rb, HH), lambda i: (i, 0)),
            pl.BlockSpec((rb, HH), lambda i: (i, 0)),
            pl.BlockSpec((1, HIDDEN), lambda i: (0, 0)),
            pl.BlockSpec((H2, HIDDEN), lambda i: (0, 0)),
            pl.BlockSpec((1, H2), lambda i: (0, 0)),
            pl.BlockSpec((HIDDEN, H2), lambda i: (0, 0)),
            pl.BlockSpec((1, HIDDEN), lambda i: (0, 0)),
        ],
        out_specs=pl.BlockSpec((rb, HIDDEN), lambda i: (i, 0)),
        out_shape=jax.ShapeDtypeStruct((N_NODES, HIDDEN), jnp.float32),
    )(parts[0, 0], parts[0, 1], parts[1, 0], parts[1, 1], h0, h1, selfvec,
      W1, b1.reshape(1, H2), W2, b2.reshape(1, HIDDEN))


def kernel(x, edge_index, edge_attr, prelu_a, W_enc, emb1, emb2, W1, b1, W2,
           b2):
    src = edge_index[0].astype(jnp.int32)
    dst = edge_index[1].astype(jnp.int32)
    cmb = (edge_attr[:, 0] * 3 + edge_attr[:, 1]).astype(jnp.int32)

    pi = jnp.arange(NPADE, dtype=jnp.int32)[None, :]
    wi = jnp.arange(NW, dtype=jnp.int32)[:, None]
    pad_src = (pi * 89 + wi * 313) % N_NODES
    pad_dst = N_NODES + (pi + wi * 7) % (ACC_ROWS - N_NODES)
    pad_cmb = (pi + wi) % 16

    src3 = jnp.concatenate([src.reshape(NW, EPW), pad_src], axis=1)
    dst3 = jnp.concatenate([dst.reshape(NW, EPW), pad_dst], axis=1)
    cmb3 = jnp.concatenate([cmb.reshape(NW, EPW), pad_cmb], axis=1)
    src3 = src3.reshape(NW, NB, B)
    dst3 = dst3.reshape(NW, NB, B)
    cmb3 = cmb3.reshape(NW, NB, B)

    emb9 = (emb1[:3, None, :] + emb2[None, :3, :]).reshape(9, HIDDEN)
    emb16 = jnp.concatenate([emb9, jnp.zeros((7, HIDDEN), jnp.float32)],
                            axis=0)
    emb2x16 = jnp.stack([emb16[:, :HH], emb16[:, HH:]], axis=0)
    selfvec = (emb1[4] + emb2[0]).reshape(1, HIDDEN)

    h0, h1 = _encode(x, prelu_a, W_enc)
    parts = _aggregate(h0, h1, src3, dst3, cmb3, emb2x16)
    return _mlp(parts, h0, h1, selfvec, W1, b1, W2, b2)
